# baseline (speedup 1.0000x reference)
import functools

import jax
import jax.numpy as jnp
from jax import lax
from jax.experimental import pallas as pl
from jax.experimental.pallas import tpu as pltpu
from jax.experimental.pallas import tpu_sc as plsc

_NC, _NS, _L = 2, 16, 16
_NW = _NC * _NS
_RC = 16
_NBUF = 3
_UNROLL = 8


def _sc_body(batch, seq, dim, x_hbm, pos_hbm, out_hbm,
             pos_buf, b0, b1, b2, si0, si1, si2, so0, so1, so2):
    bufs = (b0, b1, b2)
    sin = (si0, si1, si2)
    sout = (so0, so1, so2)
    seq_per_w = seq // _NW
    chunks_per_b = seq_per_w // _RC
    n_chunks = batch * chunks_per_b
    chunk_elems = _RC * dim
    wid = lax.axis_index("s") * _NC + lax.axis_index("c")
    seq_off = wid * (seq_per_w * dim)

    pltpu.sync_copy(pos_hbm.at[pl.ds(seq_off, seq_per_w * dim)], pos_buf)

    def chunk_off(k):
        b, cc = divmod(k, chunks_per_b)
        return b * (seq * dim) + seq_off + cc * chunk_elems

    def add_chunk(k, s):
        pos_base = (k % chunks_per_b) * chunk_elems
        buf = bufs[s]

        @plsc.parallel_loop(0, chunk_elems, step=_L, unroll=_UNROLL)
        def body(i):
            p = pos_buf[pl.ds(pos_base + i, _L)]
            plsc.addupdate(buf.at[pl.ds(i, _L)], p)

    in_d = [None] * _NBUF
    out_d = [None] * _NBUF
    for k in range(n_chunks + 1):
        if k < n_chunks:
            s = k % _NBUF
            if out_d[s] is not None:
                out_d[s].wait()
            in_d[s] = pltpu.async_copy(
                x_hbm.at[pl.ds(chunk_off(k), chunk_elems)], bufs[s], sin[s])
        if k >= 1:
            kk = k - 1
            s = kk % _NBUF
            in_d[s].wait()
            add_chunk(kk, s)
            out_d[s] = pltpu.async_copy(
                bufs[s], out_hbm.at[pl.ds(chunk_off(kk), chunk_elems)], sout[s])
    for s in range(_NBUF):
        if out_d[s] is not None:
            out_d[s].wait()


def _sc_add(x, pos_flat, batch, seq, dim):
    call = pl.kernel(
        functools.partial(_sc_body, batch, seq, dim),
        out_type=jax.ShapeDtypeStruct(x.shape, x.dtype),
        mesh=plsc.VectorSubcoreMesh(core_axis_name="c", subcore_axis_name="s"),
        scratch_types=(
            [pltpu.VMEM((seq // _NW * dim,), jnp.float32)]
            + [pltpu.VMEM((_RC * dim,), jnp.float32)] * _NBUF
            + [pltpu.SemaphoreType.DMA] * (2 * _NBUF)
        ),
    )
    return call(x, pos_flat)


def kernel(inputs, pos_table):
    batch, seq, dim = inputs.shape
    x = inputs.reshape(batch * seq * dim)
    out = _sc_add(x, pos_table.reshape(seq * dim), batch, seq, dim)
    return out.reshape(batch, seq, dim)


# speedup vs baseline: 2.4343x; 2.4343x over previous
import functools

import jax
import jax.numpy as jnp
from jax import lax
from jax.experimental import pallas as pl
from jax.experimental.pallas import tpu as pltpu
from jax.experimental.pallas import tpu_sc as plsc

_NC, _NS, _L = 2, 16, 16
_NW = _NC * _NS
_RC = 16
_NBUF = 3


def _sc_body(batch, seq, dim, x_hbm, pos_hbm, out_hbm,
             pos_buf, b0, b1, b2, si0, si1, si2, so0, so1, so2):
    bufs = (b0, b1, b2)
    sin = (si0, si1, si2)
    sout = (so0, so1, so2)
    seq_per_w = seq // _NW
    chunks_per_b = seq_per_w // _RC
    n_chunks = batch * chunks_per_b
    wid = lax.axis_index("s") * _NC + lax.axis_index("c")
    seq0 = wid * seq_per_w

    pltpu.sync_copy(pos_hbm.at[pl.ds(seq0, seq_per_w)], pos_buf)

    def chunk_row(k):
        b, cc = divmod(k, chunks_per_b)
        return b * seq + seq0 + cc * _RC

    def add_chunk(k, s):
        row_base = (k % chunks_per_b) * _RC
        buf = bufs[s]

        @plsc.parallel_loop(0, _RC)
        def body(r):
            @plsc.parallel_loop(0, dim, step=_L, unroll=4)
            def cols(c):
                p = pos_buf[row_base + r, pl.ds(c, _L)]
                plsc.addupdate(buf.at[r, pl.ds(c, _L)], p)

    in_d = [None] * _NBUF
    out_d = [None] * _NBUF
    for k in range(n_chunks + 1):
        if k < n_chunks:
            s = k % _NBUF
            if out_d[s] is not None:
                out_d[s].wait()
            in_d[s] = pltpu.async_copy(
                x_hbm.at[pl.ds(chunk_row(k), _RC)], bufs[s], sin[s])
        if k >= 1:
            kk = k - 1
            s = kk % _NBUF
            in_d[s].wait()
            add_chunk(kk, s)
            out_d[s] = pltpu.async_copy(
                bufs[s], out_hbm.at[pl.ds(chunk_row(kk), _RC)], sout[s])
    for s in range(_NBUF):
        if out_d[s] is not None:
            out_d[s].wait()


def _sc_add(x, pos_table, batch, seq, dim):
    call = pl.kernel(
        functools.partial(_sc_body, batch, seq, dim),
        out_type=jax.ShapeDtypeStruct(x.shape, x.dtype),
        mesh=plsc.VectorSubcoreMesh(core_axis_name="c", subcore_axis_name="s"),
        scratch_types=(
            [pltpu.VMEM((seq // _NW, dim), jnp.float32)]
            + [pltpu.VMEM((_RC, dim), jnp.float32)] * _NBUF
            + [pltpu.SemaphoreType.DMA] * (2 * _NBUF)
        ),
        compiler_params=pltpu.CompilerParams(use_tc_tiling_on_sc=True),
    )
    return call(x, pos_table)


def kernel(inputs, pos_table):
    batch, seq, dim = inputs.shape
    x = inputs.reshape(batch * seq, dim)
    out = _sc_add(x, pos_table, batch, seq, dim)
    return out.reshape(batch, seq, dim)
